# Initial kernel scaffold; baseline (speedup 1.0000x reference)
import jax
import jax.numpy as jnp
from jax.experimental import pallas as pl


def kernel(x, edge_index, W1, b1, W2, b2, W3, b3):
    raise NotImplementedError("write your pallas kernel here")


# speedup vs baseline: 14.9187x; 14.9187x over previous
import functools

import jax
import jax.numpy as jnp
from jax import lax
from jax.experimental import pallas as pl
from jax.experimental.pallas import tpu as pltpu
from jax.experimental.pallas import tpu_sc as plsc

F32 = jnp.float32
NC, NS, L = 2, 16, 16
NW = NC * NS
CHUNK = 128


def _mesh():
    return plsc.VectorSubcoreMesh(core_axis_name="c", subcore_axis_name="s",
                                  num_cores=NC, num_subcores=NS)


def _worker_ids():
    c = lax.axis_index("c")
    s = lax.axis_index("s")
    return c, s, s * NC + c


def _fill(ref, start, n, value):
    v = jnp.full((L,), value, F32)
    for k in range(n // L):
        ref[pl.ds(start + k * L, L)] = v


def _deg_body(nch, rps, dst_hbm, out_hbm, didx, ones, zb, acc):
    c, s, w = _worker_ids()
    pltpu.sync_copy(dst_hbm.at[w], didx)
    _fill(ones, 0, CHUNK, 1.0)
    _fill(zb, 0, rps, 0.0)
    pltpu.sync_copy(zb, acc.at[pl.ds(s * rps, rps)])
    plsc.subcore_barrier()

    def body(j, carry):
        pltpu.sync_copy(ones, acc.at[didx.at[j]], add=True)
        return carry

    lax.fori_loop(0, nch, body, 0)
    plsc.subcore_barrier()
    pltpu.sync_copy(acc.at[pl.ds(s * rps, rps)], out_hbm.at[c, pl.ds(s * rps, rps)])


def _deg_call(dstp, npad, nch):
    rps = npad // NS
    fn = pl.kernel(
        functools.partial(_deg_body, nch, rps),
        out_type=jax.ShapeDtypeStruct((NC, npad), F32),
        mesh=_mesh(),
        scratch_types=[
            pltpu.VMEM((nch, CHUNK), jnp.int32),
            pltpu.VMEM((CHUNK,), F32),
            pltpu.VMEM((rps,), F32),
            pltpu.VMEM_SHARED((npad,), F32),
        ],
    )
    return fn(dstp)


def _prop_body(nch, rps, d, hn_hbm, src_hbm, dst_hbm, out_hbm,
               sidx, didx, rbuf, acc):
    c, s, w = _worker_ids()
    pltpu.sync_copy(src_hbm.at[w], sidx)
    pltpu.sync_copy(dst_hbm.at[w], didx)

    zv = jnp.zeros((L,), F32)

    def zrow(r, carry):
        for k in range(d // L):
            rbuf[0, r, pl.ds(k * L, L)] = zv
        return carry

    lax.fori_loop(0, CHUNK, zrow, 0)
    for t in range(rps // CHUNK):
        pltpu.sync_copy(rbuf.at[0], acc.at[pl.ds(s * rps + t * CHUNK, CHUNK), :])
    plsc.subcore_barrier()

    def body(j, carry):
        pltpu.sync_copy(hn_hbm.at[sidx.at[j]], rbuf.at[0])
        pltpu.sync_copy(rbuf.at[0], acc.at[didx.at[j]], add=True)
        return carry

    lax.fori_loop(0, nch, body, 0)
    plsc.subcore_barrier()
    pltpu.sync_copy(acc.at[pl.ds(s * rps, rps), :],
                    out_hbm.at[c, pl.ds(s * rps, rps), :])


def _prop_call(hn, srcp, dstp, npad, nch, d):
    rps = npad // NS
    fn = pl.kernel(
        functools.partial(_prop_body, nch, rps, d),
        out_type=jax.ShapeDtypeStruct((NC, npad, d), F32),
        mesh=_mesh(),
        scratch_types=[
            pltpu.VMEM((nch, CHUNK), jnp.int32),
            pltpu.VMEM((nch, CHUNK), jnp.int32),
            pltpu.VMEM((1, CHUNK, d), F32),
            pltpu.VMEM_SHARED((npad, d), F32),
        ],
    )
    return fn(hn, srcp, dstp)


def _sprop_body(nch, rps, y_hbm, src_hbm, dst_hbm, out_hbm,
                sidx, didx, vbuf, acc):
    c, s, w = _worker_ids()
    pltpu.sync_copy(src_hbm.at[w], sidx)
    pltpu.sync_copy(dst_hbm.at[w], didx)
    _fill(vbuf, 0, CHUNK, 0.0)
    for t in range(rps // CHUNK):
        pltpu.sync_copy(vbuf, acc.at[pl.ds(s * rps + t * CHUNK, CHUNK)])
    plsc.subcore_barrier()

    def body(j, carry):
        pltpu.sync_copy(y_hbm.at[sidx.at[j]], vbuf)
        pltpu.sync_copy(vbuf, acc.at[didx.at[j]], add=True)
        return carry

    lax.fori_loop(0, nch, body, 0)
    plsc.subcore_barrier()
    pltpu.sync_copy(acc.at[pl.ds(s * rps, rps)], out_hbm.at[c, pl.ds(s * rps, rps)])


def _sprop_call(y, srcp, dstp, npad, nch):
    rps = npad // NS
    fn = pl.kernel(
        functools.partial(_sprop_body, nch, rps),
        out_type=jax.ShapeDtypeStruct((NC, npad), F32),
        mesh=_mesh(),
        scratch_types=[
            pltpu.VMEM((nch, CHUNK), jnp.int32),
            pltpu.VMEM((nch, CHUNK), jnp.int32),
            pltpu.VMEM((CHUNK,), F32),
            pltpu.VMEM_SHARED((npad,), F32),
        ],
    )
    return fn(y, srcp, dstp)


def _tc_prep_body(deg_ref, x_ref, w_ref, hn_ref, dis_ref):
    dis = lax.rsqrt(deg_ref[0] + deg_ref[1] + 1.0)
    dis_ref[...] = dis
    hn_ref[...] = jnp.dot(x_ref[...], w_ref[...],
                          preferred_element_type=F32) * dis


def _tc_prep_call(deg3, xpad, w1, npad, d, h, br=1024):
    g = npad // br
    return pl.pallas_call(
        _tc_prep_body,
        grid=(g,),
        in_specs=[
            pl.BlockSpec((NC, br, 1), lambda i: (0, i, 0)),
            pl.BlockSpec((br, d), lambda i: (i, 0)),
            pl.BlockSpec((d, h), lambda i: (0, 0)),
        ],
        out_specs=[
            pl.BlockSpec((br, h), lambda i: (i, 0)),
            pl.BlockSpec((br, 1), lambda i: (i, 0)),
        ],
        out_shape=[
            jax.ShapeDtypeStruct((npad, h), F32),
            jax.ShapeDtypeStruct((npad, 1), F32),
        ],
    )(deg3, xpad, w1)


def _tc_layer_body(acc_ref, hn_ref, dis_ref, b_ref, w_ref, out_ref):
    a = acc_ref[0] + acc_ref[1] + hn_ref[...]
    hact = jnp.maximum(a * dis_ref[...] + b_ref[...], 0.0)
    out_ref[...] = jnp.dot(hact, w_ref[...],
                           preferred_element_type=F32) * dis_ref[...]


def _tc_layer_call(acc, hn, dis, b, w, npad, h, hout, br=1024):
    g = npad // br
    return pl.pallas_call(
        _tc_layer_body,
        grid=(g,),
        in_specs=[
            pl.BlockSpec((NC, br, h), lambda i: (0, i, 0)),
            pl.BlockSpec((br, h), lambda i: (i, 0)),
            pl.BlockSpec((br, 1), lambda i: (i, 0)),
            pl.BlockSpec((1, h), lambda i: (0, 0)),
            pl.BlockSpec((h, hout), lambda i: (0, 0)),
        ],
        out_specs=pl.BlockSpec((br, hout), lambda i: (i, 0)),
        out_shape=jax.ShapeDtypeStruct((npad, hout), F32),
    )(acc, hn, dis, b, w)


def _tc_final_body(acc_ref, y_ref, dis_ref, b_ref, out_ref):
    out_ref[...] = ((acc_ref[0] + acc_ref[1] + y_ref[...]) * dis_ref[...]
                    + b_ref[...])


def _tc_final_call(acc3, y2d, dis2d, b3, rows):
    return pl.pallas_call(
        _tc_final_body,
        grid=(1,),
        in_specs=[
            pl.BlockSpec((NC, rows, 128), lambda i: (0, 0, 0)),
            pl.BlockSpec((rows, 128), lambda i: (0, 0)),
            pl.BlockSpec((rows, 128), lambda i: (0, 0)),
            pl.BlockSpec((1, 1), lambda i: (0, 0)),
        ],
        out_specs=pl.BlockSpec((rows, 128), lambda i: (0, 0)),
        out_shape=jax.ShapeDtypeStruct((rows, 128), F32),
    )(acc3, y2d, dis2d, b3)


def kernel(x, edge_index, W1, b1, W2, b2, W3, b3):
    n, d = x.shape
    h = W1.shape[1]
    e = edge_index.shape[1]

    npad = -(-(n + 1) // (NS * CHUNK)) * (NS * CHUNK)
    nch = -(-e // (NW * CHUNK))
    epad = NW * nch * CHUNK

    src = edge_index[0]
    dst = edge_index[1]
    padv = jnp.full((epad - e,), n, jnp.int32)
    srcp = jnp.concatenate([src, padv]).reshape(NW, nch, CHUNK)
    dstp = jnp.concatenate([dst, padv]).reshape(NW, nch, CHUNK)
    xpad = jnp.pad(x, ((0, npad - n), (0, 0)))

    deg = _deg_call(dstp, npad, nch)
    hn1, dis = _tc_prep_call(deg.reshape(NC, npad, 1), xpad, W1, npad, d, h)
    acc1 = _prop_call(hn1, srcp, dstp, npad, nch, h)
    hn2 = _tc_layer_call(acc1, hn1, dis, b1.reshape(1, h), W2, npad, h, h)
    acc2 = _prop_call(hn2, srcp, dstp, npad, nch, h)
    y = _tc_layer_call(acc2, hn2, dis, b2.reshape(1, h), W3, npad, h, 1)
    yf = y.reshape(npad)
    acc3 = _sprop_call(yf, srcp, dstp, npad, nch)

    rows = npad // 128
    out2d = _tc_final_call(acc3.reshape(NC, rows, 128), yf.reshape(rows, 128),
                           dis.reshape(rows, 128), b3.reshape(1, 1), rows)
    return out2d.reshape(npad, 1)[:n]


# speedup vs baseline: 20.4428x; 1.3703x over previous
import functools

import jax
import jax.numpy as jnp
from jax import lax
from jax.experimental import pallas as pl
from jax.experimental.pallas import tpu as pltpu
from jax.experimental.pallas import tpu_sc as plsc

F32 = jnp.float32
NC, NS, L = 2, 16, 16
NW = NC * NS
CHUNK = 80


def _mesh():
    return plsc.VectorSubcoreMesh(core_axis_name="c", subcore_axis_name="s",
                                  num_cores=NC, num_subcores=NS)


def _worker_ids():
    c = lax.axis_index("c")
    s = lax.axis_index("s")
    return c, s, s * NC + c


def _fill(ref, start, n, value):
    v = jnp.full((L,), value, F32)
    for k in range(n // L):
        ref[pl.ds(start + k * L, L)] = v


def _deg_body(nch, rps, dst_hbm, out_hbm, didx, ones, zb, acc):
    c, s, w = _worker_ids()
    pltpu.sync_copy(dst_hbm.at[w], didx)
    _fill(ones, 0, CHUNK, 1.0)
    _fill(zb, 0, rps, 0.0)
    pltpu.sync_copy(zb, acc.at[pl.ds(s * rps, rps)])
    plsc.subcore_barrier()

    def body(j, carry):
        pltpu.sync_copy(ones, acc.at[didx.at[j]], add=True)
        return carry

    lax.fori_loop(0, nch, body, 0)
    plsc.subcore_barrier()
    pltpu.sync_copy(acc.at[pl.ds(s * rps, rps)], out_hbm.at[c, pl.ds(s * rps, rps)])


def _deg_call(dstp, npad, nch):
    rps = npad // NS
    fn = pl.kernel(
        functools.partial(_deg_body, nch, rps),
        out_type=jax.ShapeDtypeStruct((NC, npad), F32),
        mesh=_mesh(),
        scratch_types=[
            pltpu.VMEM((nch, CHUNK), jnp.int32),
            pltpu.VMEM((CHUNK,), F32),
            pltpu.VMEM((rps,), F32),
            pltpu.VMEM_SHARED((npad,), F32),
        ],
    )
    return fn(dstp)


def _prop_body(nch, rps, d, hn_hbm, src_hbm, dst_hbm, out_hbm,
               sidx, didx, rbuf, acc, gsem0, gsem1):
    c, s, w = _worker_ids()
    pltpu.sync_copy(src_hbm.at[w], sidx)
    pltpu.sync_copy(dst_hbm.at[w], didx)

    zv = jnp.zeros((L,), F32)
    zrows = 64

    def zrow(r, carry):
        for k in range(d // L):
            rbuf[0, r, pl.ds(k * L, L)] = zv
        return carry

    lax.fori_loop(0, zrows, zrow, 0)
    for t in range(rps // zrows):
        pltpu.sync_copy(rbuf.at[0, pl.ds(0, zrows), :],
                        acc.at[pl.ds(s * rps + t * zrows, zrows), :])
    plsc.subcore_barrier()

    def gather(j, slot, sem):
        idx = sidx.at[pl.ds(j * CHUNK, CHUNK)]
        return pltpu.async_copy(hn_hbm.at[idx], rbuf.at[slot], sem)

    def gather_wait(j, slot, sem):
        idx = sidx.at[pl.ds(j * CHUNK, CHUNK)]
        pltpu.make_async_copy(hn_hbm.at[idx], rbuf.at[slot], sem).wait()

    gather(0, 0, gsem0)

    def body(t, carry):
        j0 = 2 * t
        j1 = j0 + 1
        gather(j1, 1, gsem1)
        gather_wait(j0, 0, gsem0)
        pltpu.sync_copy(rbuf.at[0], acc.at[didx.at[j0]], add=True)

        @pl.when(t < nch // 2 - 1)
        def _():
            gather(j1 + 1, 0, gsem0)

        gather_wait(j1, 1, gsem1)
        pltpu.sync_copy(rbuf.at[1], acc.at[didx.at[j1]], add=True)
        return carry

    lax.fori_loop(0, nch // 2, body, 0)
    plsc.subcore_barrier()
    pltpu.sync_copy(acc.at[pl.ds(s * rps, rps), :],
                    out_hbm.at[c, pl.ds(s * rps, rps), :])


def _prop_call(hn, srcp, dstp, npad, nch, d):
    rps = npad // NS
    fn = pl.kernel(
        functools.partial(_prop_body, nch, rps, d),
        out_type=jax.ShapeDtypeStruct((NC, npad, d), F32),
        mesh=_mesh(),
        scratch_types=[
            pltpu.VMEM((nch * CHUNK,), jnp.int32),
            pltpu.VMEM((nch, CHUNK), jnp.int32),
            pltpu.VMEM((2, CHUNK, d), F32),
            pltpu.VMEM_SHARED((npad, d), F32),
            pltpu.SemaphoreType.DMA,
            pltpu.SemaphoreType.DMA,
        ],
    )
    return fn(hn, srcp, dstp)


def _sprop_body(nch, rps, y_hbm, src_hbm, dst_hbm, out_hbm,
                sidx, didx, vbuf, acc, gsem0, gsem1):
    c, s, w = _worker_ids()
    pltpu.sync_copy(src_hbm.at[w], sidx)
    pltpu.sync_copy(dst_hbm.at[w], didx)
    zrows = 64
    _fill(vbuf.at[0], 0, zrows, 0.0)
    for t in range(rps // zrows):
        pltpu.sync_copy(vbuf.at[0, pl.ds(0, zrows)],
                        acc.at[pl.ds(s * rps + t * zrows, zrows)])
    plsc.subcore_barrier()

    def gather(j, slot, sem):
        idx = sidx.at[pl.ds(j * CHUNK, CHUNK)]
        return pltpu.async_copy(y_hbm.at[idx], vbuf.at[slot], sem)

    def gather_wait(j, slot, sem):
        idx = sidx.at[pl.ds(j * CHUNK, CHUNK)]
        pltpu.make_async_copy(y_hbm.at[idx], vbuf.at[slot], sem).wait()

    gather(0, 0, gsem0)

    def body(t, carry):
        j0 = 2 * t
        j1 = j0 + 1
        gather(j1, 1, gsem1)
        gather_wait(j0, 0, gsem0)
        pltpu.sync_copy(vbuf.at[0], acc.at[didx.at[j0]], add=True)

        @pl.when(t < nch // 2 - 1)
        def _():
            gather(j1 + 1, 0, gsem0)

        gather_wait(j1, 1, gsem1)
        pltpu.sync_copy(vbuf.at[1], acc.at[didx.at[j1]], add=True)
        return carry

    lax.fori_loop(0, nch // 2, body, 0)
    plsc.subcore_barrier()
    pltpu.sync_copy(acc.at[pl.ds(s * rps, rps)], out_hbm.at[c, pl.ds(s * rps, rps)])


def _sprop_call(y, srcp, dstp, npad, nch):
    rps = npad // NS
    fn = pl.kernel(
        functools.partial(_sprop_body, nch, rps),
        out_type=jax.ShapeDtypeStruct((NC, npad), F32),
        mesh=_mesh(),
        scratch_types=[
            pltpu.VMEM((nch * CHUNK,), jnp.int32),
            pltpu.VMEM((nch, CHUNK), jnp.int32),
            pltpu.VMEM((2, CHUNK), F32),
            pltpu.VMEM_SHARED((npad,), F32),
            pltpu.SemaphoreType.DMA,
            pltpu.SemaphoreType.DMA,
        ],
    )
    return fn(y, srcp, dstp)


def _tc_prep_body(deg_ref, x_ref, w_ref, hn_ref, dis_ref):
    dis = lax.rsqrt(deg_ref[0] + deg_ref[1] + 1.0)
    dis_ref[...] = dis
    hn_ref[...] = jnp.dot(x_ref[...], w_ref[...],
                          preferred_element_type=F32) * dis


def _tc_prep_call(deg3, xpad, w1, npad, d, h, br=1024):
    g = npad // br
    return pl.pallas_call(
        _tc_prep_body,
        grid=(g,),
        in_specs=[
            pl.BlockSpec((NC, br, 1), lambda i: (0, i, 0)),
            pl.BlockSpec((br, d), lambda i: (i, 0)),
            pl.BlockSpec((d, h), lambda i: (0, 0)),
        ],
        out_specs=[
            pl.BlockSpec((br, h), lambda i: (i, 0)),
            pl.BlockSpec((br, 1), lambda i: (i, 0)),
        ],
        out_shape=[
            jax.ShapeDtypeStruct((npad, h), F32),
            jax.ShapeDtypeStruct((npad, 1), F32),
        ],
    )(deg3, xpad, w1)


def _tc_layer_body(acc_ref, hn_ref, dis_ref, b_ref, w_ref, out_ref):
    a = acc_ref[0] + acc_ref[1] + hn_ref[...]
    hact = jnp.maximum(a * dis_ref[...] + b_ref[...], 0.0)
    out_ref[...] = jnp.dot(hact, w_ref[...],
                           preferred_element_type=F32) * dis_ref[...]


def _tc_layer_call(acc, hn, dis, b, w, npad, h, hout, br=1024):
    g = npad // br
    return pl.pallas_call(
        _tc_layer_body,
        grid=(g,),
        in_specs=[
            pl.BlockSpec((NC, br, h), lambda i: (0, i, 0)),
            pl.BlockSpec((br, h), lambda i: (i, 0)),
            pl.BlockSpec((br, 1), lambda i: (i, 0)),
            pl.BlockSpec((1, h), lambda i: (0, 0)),
            pl.BlockSpec((h, hout), lambda i: (0, 0)),
        ],
        out_specs=pl.BlockSpec((br, hout), lambda i: (i, 0)),
        out_shape=jax.ShapeDtypeStruct((npad, hout), F32),
    )(acc, hn, dis, b, w)


def _tc_final_body(acc_ref, y_ref, dis_ref, b_ref, out_ref):
    out_ref[...] = ((acc_ref[0] + acc_ref[1] + y_ref[...]) * dis_ref[...]
                    + b_ref[...])


def _tc_final_call(acc3, y2d, dis2d, b3, rows):
    return pl.pallas_call(
        _tc_final_body,
        grid=(1,),
        in_specs=[
            pl.BlockSpec((NC, rows, 128), lambda i: (0, 0, 0)),
            pl.BlockSpec((rows, 128), lambda i: (0, 0)),
            pl.BlockSpec((rows, 128), lambda i: (0, 0)),
            pl.BlockSpec((1, 1), lambda i: (0, 0)),
        ],
        out_specs=pl.BlockSpec((rows, 128), lambda i: (0, 0)),
        out_shape=jax.ShapeDtypeStruct((rows, 128), F32),
    )(acc3, y2d, dis2d, b3)


def kernel(x, edge_index, W1, b1, W2, b2, W3, b3):
    n, d = x.shape
    h = W1.shape[1]
    e = edge_index.shape[1]

    npad = -(-(n + 1) // (NS * 128)) * (NS * 128)
    nch = -(-e // (NW * CHUNK))
    nch += nch % 2
    epad = NW * nch * CHUNK

    src = edge_index[0]
    dst = edge_index[1]
    padv = jnp.full((epad - e,), n, jnp.int32)
    srcp = jnp.concatenate([src, padv]).reshape(NW, nch * CHUNK)
    dstp = jnp.concatenate([dst, padv]).reshape(NW, nch, CHUNK)
    xpad = jnp.pad(x, ((0, npad - n), (0, 0)))

    deg = _deg_call(dstp, npad, nch)
    hn1, dis = _tc_prep_call(deg.reshape(NC, npad, 1), xpad, W1, npad, d, h)
    acc1 = _prop_call(hn1, srcp, dstp, npad, nch, h)
    hn2 = _tc_layer_call(acc1, hn1, dis, b1.reshape(1, h), W2, npad, h, h)
    acc2 = _prop_call(hn2, srcp, dstp, npad, nch, h)
    y = _tc_layer_call(acc2, hn2, dis, b2.reshape(1, h), W3, npad, h, 1)
    yf = y.reshape(npad)
    acc3 = _sprop_call(yf, srcp, dstp, npad, nch)

    rows = npad // 128
    out2d = _tc_final_call(acc3.reshape(NC, rows, 128), yf.reshape(rows, 128),
                           dis.reshape(rows, 128), b3.reshape(1, 1), rows)
    return out2d.reshape(npad, 1)[:n]
